# Initial kernel scaffold; baseline (speedup 1.0000x reference)
import jax
import jax.numpy as jnp
from jax.experimental import pallas as pl


def kernel(atom_fea, nbr_fea, nbr_fea_idx, params):
    raise NotImplementedError("write your pallas kernel here")


# speedup vs baseline: 1.9414x; 1.9414x over previous
import functools

import jax
import jax.numpy as jnp
from jax import lax
from jax.experimental import pallas as pl
from jax.experimental.pallas import tpu as pltpu
from jax.experimental.pallas import tpu_sc as plsc

F32 = jnp.float32

N_ATOMS = 50000
MAX_NBR = 16
FEA = 64

NW = 32
CHUNK = 8
ROWS_W = 1600
NCHUNK = ROWS_W // CHUNK
N_PAD = NW * ROWS_W

ROW_BLK = 1024
N_TC_BLOCKS = N_PAD // ROW_BLK


def _round_bf16(v):
    u = plsc.bitcast(v, jnp.int32)
    r = (u + 0x7FFF + ((u >> 16) & 1)) & jnp.int32(-65536)
    return plsc.bitcast(r, F32)


def _sc_body(a_hbm, b_hbm, idx_hbm, s_hbm,
             idx_v0, idx_v1, gat_v0, gat_v1, a_v, s_v, sem0, sem1):
    cid = lax.axis_index("c")
    sid = lax.axis_index("s")
    wid = sid * 2 + cid
    base = wid * ROWS_W

    idx_v = (idx_v0, idx_v1)
    gat_v = (gat_v0, gat_v1)
    sems = (sem0, sem1)

    def load_idx(k, p):
        ebase = (base + k * CHUNK) * MAX_NBR
        pltpu.sync_copy(idx_hbm.at[pl.ds(ebase, CHUNK * MAX_NBR)], idx_v[p])

    def fire_gather(p):
        pltpu.async_copy(b_hbm.at[idx_v[p]], gat_v[p], sems[p])

    def wait_gather(p):
        pltpu.make_async_copy(b_hbm.at[idx_v[p]], gat_v[p], sems[p]).wait()

    load_idx(0, 0)
    fire_gather(0)

    def pair_body(k2, _):
        for p in range(2):
            k = k2 * 2 + p
            rbase = base + k * CHUNK
            load_idx(lax.rem(k + 1, NCHUNK), 1 - p)
            fire_gather(1 - p)
            pltpu.sync_copy(a_hbm.at[pl.ds(rbase, CHUNK)], a_v)
            wait_gather(p)
            g = gat_v[p]
            for a in range(CHUNK):
                for f in range(FEA // 16):
                    av = a_v[a, pl.ds(f * 16, 16)]
                    acc = _round_bf16(jnp.maximum(
                        av + g[a * MAX_NBR, pl.ds(f * 16, 16)], 0.0))
                    for n in range(1, MAX_NBR):
                        acc = acc + _round_bf16(jnp.maximum(
                            av + g[a * MAX_NBR + n, pl.ds(f * 16, 16)], 0.0))
                    s_v[a, pl.ds(f * 16, 16)] = acc
            pltpu.sync_copy(s_v, s_hbm.at[pl.ds(rbase, CHUNK)])
        return ()

    lax.fori_loop(0, NCHUNK // 2, pair_body, ())
    wait_gather(0)


@jax.jit
def _sc_segment_relu_sum(a_pad, b_pad, idx_flat):
    run = pl.kernel(
        _sc_body,
        out_type=jax.ShapeDtypeStruct((N_PAD, FEA), F32),
        mesh=plsc.VectorSubcoreMesh(core_axis_name="c", subcore_axis_name="s"),
        scratch_types=[
            pltpu.VMEM((CHUNK * MAX_NBR,), jnp.int32),
            pltpu.VMEM((CHUNK * MAX_NBR,), jnp.int32),
            pltpu.VMEM((CHUNK * MAX_NBR, FEA), F32),
            pltpu.VMEM((CHUNK * MAX_NBR, FEA), F32),
            pltpu.VMEM((CHUNK, FEA), F32),
            pltpu.VMEM((CHUNK, FEA), F32),
            pltpu.SemaphoreType.DMA,
            pltpu.SemaphoreType.DMA,
        ],
        compiler_params=pltpu.CompilerParams(use_tc_tiling_on_sc=False,
                                             needs_layout_passes=False),
    )
    return run(a_pad, b_pad, idx_flat)


def _full(shape):
    return pl.BlockSpec(shape, lambda i: (0, 0))


def _rows(cols):
    return pl.BlockSpec((ROW_BLK, cols), lambda i: (i, 0))


def _dot(x, w):
    return jnp.dot(x.astype(jnp.bfloat16), w.astype(jnp.bfloat16),
                   preferred_element_type=F32)


def _dot_f32(x, w):
    return jnp.dot(x, w, preferred_element_type=F32,
                   precision=lax.Precision.HIGHEST)


def _stage_embed(af_pad, we, be, w1c, b1c):
    def body(af_ref, we_ref, be_ref, w1_ref, b1_ref, x_ref, a_ref, b_ref):
        x = _dot(af_ref[...], we_ref[...]) + be_ref[...]
        x_ref[...] = x
        ab = _dot(x, w1_ref[...]) + b1_ref[...]
        a_ref[...] = ab[:, :FEA]
        b_ref[...] = ab[:, FEA:]

    return pl.pallas_call(
        body,
        grid=(N_TC_BLOCKS,),
        in_specs=[_rows(128), _full((128, FEA)), _full((1, FEA)),
                  _full((FEA, 2 * FEA)), _full((1, 2 * FEA))],
        out_specs=[_rows(FEA), _rows(FEA), _rows(FEA)],
        out_shape=[jax.ShapeDtypeStruct((N_PAD, FEA), F32),
                   jax.ShapeDtypeStruct((N_PAD, FEA), F32),
                   jax.ShapeDtypeStruct((N_PAD, FEA), F32)],
    )(af_pad, we, be, w1c, b1c)


def _update_block(x, s, w2_ref, b2s_ref, u1a_ref, u1b_ref, bu1_ref,
                  u2_ref, bu2_ref):
    agg = _dot_f32(s, w2_ref[...]) + b2s_ref[...]
    t = jax.nn.relu(_dot(x, u1a_ref[...]) + _dot(agg, u1b_ref[...])
                    + bu1_ref[...])
    upd = _dot(t, u2_ref[...]) + bu2_ref[...]
    return jax.nn.relu(x + upd)


def _stage_update(x, s, w2, b2s, u1a, u1b, bu1, u2, bu2, w1c, b1c):
    def body(x_ref, s_ref, w2_ref, b2s_ref, u1a_ref, u1b_ref, bu1_ref,
             u2_ref, bu2_ref, w1_ref, b1_ref, xn_ref, a_ref, b_ref):
        xn = _update_block(x_ref[...], s_ref[...], w2_ref, b2s_ref,
                           u1a_ref, u1b_ref, bu1_ref, u2_ref, bu2_ref)
        xn_ref[...] = xn
        ab = _dot(xn, w1_ref[...]) + b1_ref[...]
        a_ref[...] = ab[:, :FEA]
        b_ref[...] = ab[:, FEA:]

    wspec = _full((FEA, FEA))
    bspec = _full((1, FEA))
    return pl.pallas_call(
        body,
        grid=(N_TC_BLOCKS,),
        in_specs=[_rows(FEA), _rows(FEA), wspec, bspec, wspec, wspec, bspec,
                  wspec, bspec, _full((FEA, 2 * FEA)), _full((1, 2 * FEA))],
        out_specs=[_rows(FEA), _rows(FEA), _rows(FEA)],
        out_shape=[jax.ShapeDtypeStruct((N_PAD, FEA), F32),
                   jax.ShapeDtypeStruct((N_PAD, FEA), F32),
                   jax.ShapeDtypeStruct((N_PAD, FEA), F32)],
    )(x, s, w2, b2s, u1a, u1b, bu1, u2, bu2, w1c, b1c)


def _stage_final(x, s, w2, b2s, u1a, u1b, bu1, u2, bu2,
                 f1, bf1, f2, bf2, hw, bh):
    def body(x_ref, s_ref, w2_ref, b2s_ref, u1a_ref, u1b_ref, bu1_ref,
             u2_ref, bu2_ref, f1_ref, bf1_ref, f2_ref, bf2_ref,
             hw_ref, bh_ref, xn_ref, feat_ref, prop_ref):
        xn = _update_block(x_ref[...], s_ref[...], w2_ref, b2s_ref,
                           u1a_ref, u1b_ref, bu1_ref, u2_ref, bu2_ref)
        xn_ref[...] = xn
        h = jax.nn.relu(_dot(xn, f1_ref[...]) + bf1_ref[...])
        feat = _dot(h, f2_ref[...]) + bf2_ref[...]
        feat_ref[...] = feat
        prop_ref[...] = _dot(feat, hw_ref[...]) + bh_ref[...]

    wspec = _full((FEA, FEA))
    bspec = _full((1, FEA))
    return pl.pallas_call(
        body,
        grid=(N_TC_BLOCKS,),
        in_specs=[_rows(FEA), _rows(FEA), wspec, bspec, wspec, wspec, bspec,
                  wspec, bspec, _full((FEA, 2 * FEA)), _full((1, 2 * FEA)),
                  _full((2 * FEA, FEA)), _full((1, FEA)),
                  _full((FEA, 8)), _full((1, 8))],
        out_specs=[_rows(FEA), _rows(FEA), _rows(8)],
        out_shape=[jax.ShapeDtypeStruct((N_PAD, FEA), F32),
                   jax.ShapeDtypeStruct((N_PAD, FEA), F32),
                   jax.ShapeDtypeStruct((N_PAD, 8), F32)],
    )(x, s, w2, b2s, u1a, u1b, bu1, u2, bu2, f1, bf1, f2, bf2, hw, bh)


def _conv_weights(cp):
    w1, b1 = cp["msg1"]
    w2, b2 = cp["msg2"]
    u1, bu1 = cp["upd1"]
    u2, bu2 = cp["upd2"]
    w1c = jnp.concatenate([w1[:FEA], w1[FEA:]], axis=1)
    b1c = jnp.concatenate([b1, jnp.zeros((FEA,), F32)])[None, :]
    b2s = (MAX_NBR * b2)[None, :]
    w2r = w2.astype(jnp.bfloat16).astype(F32)
    return (w1c, b1c, w2r, b2s, u1[:FEA], u1[FEA:], bu1[None, :],
            u2, bu2[None, :])


def kernel(atom_fea, nbr_fea, nbr_fea_idx, params):
    del nbr_fea

    n = atom_fea.shape[0]
    pad = N_PAD - n

    af = jnp.pad(atom_fea, ((0, pad), (0, 128 - atom_fea.shape[1])))
    idx_flat = jnp.pad(nbr_fea_idx, ((0, pad), (0, 0))).reshape(-1)

    we, be = params["atom_emb"]
    we = jnp.pad(we, ((0, 128 - we.shape[0]), (0, 0)))
    conv_ws = [_conv_weights(cp) for cp in params["convs"]]

    f1, bf1 = params["feat1"]
    f2, bf2 = params["feat2"]
    head_w = jnp.concatenate(
        [params[nm][0] for nm in
         ("bond_strength", "local_charge", "coordination_energy",
          "migration_barrier")], axis=1)
    head_b = jnp.stack(
        [params[nm][1][0] for nm in
         ("bond_strength", "local_charge", "coordination_energy",
          "migration_barrier")])
    head_w = jnp.pad(head_w, ((0, 0), (0, 4)))
    head_b = jnp.pad(head_b, (0, 4))[None, :]

    w1c0, b1c0 = conv_ws[0][0], conv_ws[0][1]
    x, a_arr, b_arr = _stage_embed(af, we, be[None, :], w1c0, b1c0)

    for c in range(3):
        (w1c, b1c, w2, b2s, u1a, u1b, bu1, u2, bu2) = conv_ws[c]
        s = _sc_segment_relu_sum(a_arr, b_arr, idx_flat)
        if c < 2:
            nw1c, nb1c = conv_ws[c + 1][0], conv_ws[c + 1][1]
            x, a_arr, b_arr = _stage_update(x, s, w2, b2s, u1a, u1b, bu1,
                                            u2, bu2, nw1c, nb1c)
        else:
            x, feat, props = _stage_final(x, s, w2, b2s, u1a, u1b, bu1,
                                          u2, bu2, f1, bf1[None, :],
                                          f2, bf2[None, :], head_w, head_b)

    props = props[:n]
    return (props[:, 0:1], props[:, 1:2], props[:, 2:3], props[:, 3:4],
            feat[:n], x[:n])


# speedup vs baseline: 1.9779x; 1.0188x over previous
import functools

import jax
import jax.numpy as jnp
from jax import lax
from jax.experimental import pallas as pl
from jax.experimental.pallas import tpu as pltpu
from jax.experimental.pallas import tpu_sc as plsc

F32 = jnp.float32

N_ATOMS = 50000
MAX_NBR = 16
FEA = 64

NW = 32
CHUNK = 8
ROWS_W = 1600
NCHUNK = ROWS_W // CHUNK
N_PAD = NW * ROWS_W

ROW_BLK = 1024
N_TC_BLOCKS = N_PAD // ROW_BLK


def _round_bf16(v):
    u = plsc.bitcast(v, jnp.int32)
    r = (u + 0x7FFF + ((u >> 16) & 1)) & jnp.int32(-65536)
    return plsc.bitcast(r, F32)


def _sc_body(a_hbm, b_hbm, idx2_hbm, s_hbm,
             idx_v0, idx_v1, gat_v0, gat_v1, a_v0, a_v1, s_v0, s_v1,
             isem0, isem1, gsem0, gsem1, asem0, asem1, ssem0, ssem1):
    cid = lax.axis_index("c")
    sid = lax.axis_index("s")
    wid = sid * 2 + cid
    base = wid * ROWS_W
    irow0 = wid * NCHUNK

    idx_v = (idx_v0, idx_v1)
    gat_v = (gat_v0, gat_v1)
    a_v = (a_v0, a_v1)
    s_v = (s_v0, s_v1)
    isem = (isem0, isem1)
    gsem = (gsem0, gsem1)
    asem = (asem0, asem1)
    ssem = (ssem0, ssem1)

    def i_copy(k, p):
        return pltpu.make_async_copy(idx2_hbm.at[irow0 + k], idx_v[p], isem[p])

    def g_copy(p):
        return pltpu.make_async_copy(b_hbm.at[idx_v[p]], gat_v[p], gsem[p])

    def a_copy(k, p):
        return pltpu.make_async_copy(a_hbm.at[pl.ds(base + k * CHUNK, CHUNK)],
                                     a_v[p], asem[p])

    def s_copy(k, p):
        return pltpu.make_async_copy(s_v[p],
                                     s_hbm.at[pl.ds(base + k * CHUNK, CHUNK)],
                                     ssem[p])

    i_copy(0, 0).start()
    i_copy(1, 1).start()
    a_copy(0, 0).start()
    a_copy(1, 1).start()
    i_copy(0, 0).wait()
    g_copy(0).start()

    def pair_body(k2, _):
        for q in range(2):
            k = k2 * 2 + q
            g_copy(q).wait()

            @pl.when(k + 2 < NCHUNK)
            def _():
                i_copy(k + 2, q).start()

            @pl.when(k + 1 < NCHUNK)
            def _():
                i_copy(k + 1, 1 - q).wait()
                g_copy(1 - q).start()

            a_copy(k, q).wait()

            @pl.when(k >= 2)
            def _():
                s_copy(k - 2, q).wait()

            g = gat_v[q]
            av_ref = a_v[q]
            sv_ref = s_v[q]
            for a in range(CHUNK):
                for f in range(FEA // 16):
                    av = av_ref[a, pl.ds(f * 16, 16)]
                    acc = _round_bf16(jnp.maximum(
                        av + g[a * MAX_NBR, pl.ds(f * 16, 16)], 0.0))
                    for n in range(1, MAX_NBR):
                        acc = acc + _round_bf16(jnp.maximum(
                            av + g[a * MAX_NBR + n, pl.ds(f * 16, 16)], 0.0))
                    sv_ref[a, pl.ds(f * 16, 16)] = acc
            s_copy(k, q).start()

            @pl.when(k + 2 < NCHUNK)
            def _():
                a_copy(k + 2, q).start()
        return ()

    lax.fori_loop(0, NCHUNK // 2, pair_body, ())
    s_copy(NCHUNK - 2, 0).wait()
    s_copy(NCHUNK - 1, 1).wait()


@jax.jit
def _sc_segment_relu_sum(a_pad, b_pad, idx2):
    run = pl.kernel(
        _sc_body,
        out_type=jax.ShapeDtypeStruct((N_PAD, FEA), F32),
        mesh=plsc.VectorSubcoreMesh(core_axis_name="c", subcore_axis_name="s"),
        scratch_types=[
            pltpu.VMEM((CHUNK * MAX_NBR,), jnp.int32),
            pltpu.VMEM((CHUNK * MAX_NBR,), jnp.int32),
            pltpu.VMEM((CHUNK * MAX_NBR, FEA), F32),
            pltpu.VMEM((CHUNK * MAX_NBR, FEA), F32),
            pltpu.VMEM((CHUNK, FEA), F32),
            pltpu.VMEM((CHUNK, FEA), F32),
            pltpu.VMEM((CHUNK, FEA), F32),
            pltpu.VMEM((CHUNK, FEA), F32),
        ] + [pltpu.SemaphoreType.DMA] * 8,
        compiler_params=pltpu.CompilerParams(use_tc_tiling_on_sc=False,
                                             needs_layout_passes=False),
    )
    return run(a_pad, b_pad, idx2)


def _full(shape):
    return pl.BlockSpec(shape, lambda i: (0, 0))


def _rows(cols):
    return pl.BlockSpec((ROW_BLK, cols), lambda i: (i, 0))


def _dot(x, w):
    return jnp.dot(x.astype(jnp.bfloat16), w.astype(jnp.bfloat16),
                   preferred_element_type=F32)


def _dot_f32(x, w):
    return jnp.dot(x, w, preferred_element_type=F32,
                   precision=lax.Precision.HIGHEST)


def _stage_embed(af_pad, we, be, w1c, b1c):
    def body(af_ref, we_ref, be_ref, w1_ref, b1_ref, x_ref, a_ref, b_ref):
        x = _dot(af_ref[...], we_ref[...]) + be_ref[...]
        x_ref[...] = x
        ab = _dot(x, w1_ref[...]) + b1_ref[...]
        a_ref[...] = ab[:, :FEA]
        b_ref[...] = ab[:, FEA:]

    return pl.pallas_call(
        body,
        grid=(N_TC_BLOCKS,),
        in_specs=[_rows(128), _full((128, FEA)), _full((1, FEA)),
                  _full((FEA, 2 * FEA)), _full((1, 2 * FEA))],
        out_specs=[_rows(FEA), _rows(FEA), _rows(FEA)],
        out_shape=[jax.ShapeDtypeStruct((N_PAD, FEA), F32),
                   jax.ShapeDtypeStruct((N_PAD, FEA), F32),
                   jax.ShapeDtypeStruct((N_PAD, FEA), F32)],
    )(af_pad, we, be, w1c, b1c)


def _update_block(x, s, w2_ref, b2s_ref, u1a_ref, u1b_ref, bu1_ref,
                  u2_ref, bu2_ref):
    agg = _dot_f32(s, w2_ref[...]) + b2s_ref[...]
    t = jax.nn.relu(_dot(x, u1a_ref[...]) + _dot(agg, u1b_ref[...])
                    + bu1_ref[...])
    upd = _dot(t, u2_ref[...]) + bu2_ref[...]
    return jax.nn.relu(x + upd)


def _stage_update(x, s, w2, b2s, u1a, u1b, bu1, u2, bu2, w1c, b1c):
    def body(x_ref, s_ref, w2_ref, b2s_ref, u1a_ref, u1b_ref, bu1_ref,
             u2_ref, bu2_ref, w1_ref, b1_ref, xn_ref, a_ref, b_ref):
        xn = _update_block(x_ref[...], s_ref[...], w2_ref, b2s_ref,
                           u1a_ref, u1b_ref, bu1_ref, u2_ref, bu2_ref)
        xn_ref[...] = xn
        ab = _dot(xn, w1_ref[...]) + b1_ref[...]
        a_ref[...] = ab[:, :FEA]
        b_ref[...] = ab[:, FEA:]

    wspec = _full((FEA, FEA))
    bspec = _full((1, FEA))
    return pl.pallas_call(
        body,
        grid=(N_TC_BLOCKS,),
        in_specs=[_rows(FEA), _rows(FEA), wspec, bspec, wspec, wspec, bspec,
                  wspec, bspec, _full((FEA, 2 * FEA)), _full((1, 2 * FEA))],
        out_specs=[_rows(FEA), _rows(FEA), _rows(FEA)],
        out_shape=[jax.ShapeDtypeStruct((N_PAD, FEA), F32),
                   jax.ShapeDtypeStruct((N_PAD, FEA), F32),
                   jax.ShapeDtypeStruct((N_PAD, FEA), F32)],
    )(x, s, w2, b2s, u1a, u1b, bu1, u2, bu2, w1c, b1c)


def _stage_final(x, s, w2, b2s, u1a, u1b, bu1, u2, bu2,
                 f1, bf1, f2, bf2, hw, bh):
    def body(x_ref, s_ref, w2_ref, b2s_ref, u1a_ref, u1b_ref, bu1_ref,
             u2_ref, bu2_ref, f1_ref, bf1_ref, f2_ref, bf2_ref,
             hw_ref, bh_ref, xn_ref, feat_ref, prop_ref):
        xn = _update_block(x_ref[...], s_ref[...], w2_ref, b2s_ref,
                           u1a_ref, u1b_ref, bu1_ref, u2_ref, bu2_ref)
        xn_ref[...] = xn
        h = jax.nn.relu(_dot(xn, f1_ref[...]) + bf1_ref[...])
        feat = _dot(h, f2_ref[...]) + bf2_ref[...]
        feat_ref[...] = feat
        prop_ref[...] = _dot(feat, hw_ref[...]) + bh_ref[...]

    wspec = _full((FEA, FEA))
    bspec = _full((1, FEA))
    return pl.pallas_call(
        body,
        grid=(N_TC_BLOCKS,),
        in_specs=[_rows(FEA), _rows(FEA), wspec, bspec, wspec, wspec, bspec,
                  wspec, bspec, _full((FEA, 2 * FEA)), _full((1, 2 * FEA)),
                  _full((2 * FEA, FEA)), _full((1, FEA)),
                  _full((FEA, 8)), _full((1, 8))],
        out_specs=[_rows(FEA), _rows(FEA), _rows(8)],
        out_shape=[jax.ShapeDtypeStruct((N_PAD, FEA), F32),
                   jax.ShapeDtypeStruct((N_PAD, FEA), F32),
                   jax.ShapeDtypeStruct((N_PAD, 8), F32)],
    )(x, s, w2, b2s, u1a, u1b, bu1, u2, bu2, f1, bf1, f2, bf2, hw, bh)


def _conv_weights(cp):
    w1, b1 = cp["msg1"]
    w2, b2 = cp["msg2"]
    u1, bu1 = cp["upd1"]
    u2, bu2 = cp["upd2"]
    w1c = jnp.concatenate([w1[:FEA], w1[FEA:]], axis=1)
    b1c = jnp.concatenate([b1, jnp.zeros((FEA,), F32)])[None, :]
    b2s = (MAX_NBR * b2)[None, :]
    w2r = w2.astype(jnp.bfloat16).astype(F32)
    return (w1c, b1c, w2r, b2s, u1[:FEA], u1[FEA:], bu1[None, :],
            u2, bu2[None, :])


def kernel(atom_fea, nbr_fea, nbr_fea_idx, params):
    del nbr_fea

    n = atom_fea.shape[0]
    pad = N_PAD - n

    af = jnp.pad(atom_fea, ((0, pad), (0, 128 - atom_fea.shape[1])))
    idx2 = jnp.pad(nbr_fea_idx, ((0, pad), (0, 0))).reshape(
        N_PAD * MAX_NBR // (CHUNK * MAX_NBR), CHUNK * MAX_NBR)

    we, be = params["atom_emb"]
    we = jnp.pad(we, ((0, 128 - we.shape[0]), (0, 0)))
    conv_ws = [_conv_weights(cp) for cp in params["convs"]]

    f1, bf1 = params["feat1"]
    f2, bf2 = params["feat2"]
    head_w = jnp.concatenate(
        [params[nm][0] for nm in
         ("bond_strength", "local_charge", "coordination_energy",
          "migration_barrier")], axis=1)
    head_b = jnp.stack(
        [params[nm][1][0] for nm in
         ("bond_strength", "local_charge", "coordination_energy",
          "migration_barrier")])
    head_w = jnp.pad(head_w, ((0, 0), (0, 4)))
    head_b = jnp.pad(head_b, (0, 4))[None, :]

    w1c0, b1c0 = conv_ws[0][0], conv_ws[0][1]
    x, a_arr, b_arr = _stage_embed(af, we, be[None, :], w1c0, b1c0)

    for c in range(3):
        (w1c, b1c, w2, b2s, u1a, u1b, bu1, u2, bu2) = conv_ws[c]
        s = _sc_segment_relu_sum(a_arr, b_arr, idx2)
        if c < 2:
            nw1c, nb1c = conv_ws[c + 1][0], conv_ws[c + 1][1]
            x, a_arr, b_arr = _stage_update(x, s, w2, b2s, u1a, u1b, bu1,
                                            u2, bu2, nw1c, nb1c)
        else:
            x, feat, props = _stage_final(x, s, w2, b2s, u1a, u1b, bu1,
                                          u2, bu2, f1, bf1[None, :],
                                          f2, bf2[None, :], head_w, head_b)

    props = props[:n]
    return (props[:, 0:1], props[:, 1:2], props[:, 2:3], props[:, 3:4],
            feat[:n], x[:n])


# speedup vs baseline: 2.1385x; 1.0812x over previous
import functools

import jax
import jax.numpy as jnp
from jax import lax
from jax.experimental import pallas as pl
from jax.experimental.pallas import tpu as pltpu
from jax.experimental.pallas import tpu_sc as plsc

F32 = jnp.float32

N_ATOMS = 50000
MAX_NBR = 16
FEA = 64

NW = 32
CHUNK = 8
ROWS_C0 = 2048
ROWS_C1 = 1152
N_PAD = 16 * (ROWS_C0 + ROWS_C1)

ROW_BLK = 1024
N_TC_BLOCKS = N_PAD // ROW_BLK


def _round_bf16(v):
    u = plsc.bitcast(v, jnp.int32)
    r = (u + 0x7FFF + ((u >> 16) & 1)) & jnp.int32(-65536)
    return plsc.bitcast(r, F32)


def _sc_body(a_hbm, b_hbm, idx2_hbm, s_hbm,
             idx_v0, idx_v1, gat_v0, gat_v1, a_v0, a_v1, s_v0, s_v1,
             isem0, isem1, gsem0, gsem1, asem0, asem1, ssem0, ssem1):
    cid = lax.axis_index("c")
    sid = lax.axis_index("s")
    base = lax.select(cid == 0, sid * ROWS_C0,
                      16 * ROWS_C0 + sid * ROWS_C1)
    nchunk = lax.select(cid == 0, ROWS_C0 // CHUNK, ROWS_C1 // CHUNK)
    irow0 = base // CHUNK

    idx_v = (idx_v0, idx_v1)
    gat_v = (gat_v0, gat_v1)
    a_v = (a_v0, a_v1)
    s_v = (s_v0, s_v1)
    isem = (isem0, isem1)
    gsem = (gsem0, gsem1)
    asem = (asem0, asem1)
    ssem = (ssem0, ssem1)

    def i_copy(k, p):
        return pltpu.make_async_copy(idx2_hbm.at[irow0 + k], idx_v[p], isem[p])

    def g_copy(p):
        return pltpu.make_async_copy(b_hbm.at[idx_v[p]], gat_v[p], gsem[p])

    def a_copy(k, p):
        return pltpu.make_async_copy(a_hbm.at[pl.ds(base + k * CHUNK, CHUNK)],
                                     a_v[p], asem[p])

    def s_copy(k, p):
        return pltpu.make_async_copy(s_v[p],
                                     s_hbm.at[pl.ds(base + k * CHUNK, CHUNK)],
                                     ssem[p])

    i_copy(0, 0).start()
    i_copy(1, 1).start()
    a_copy(0, 0).start()
    a_copy(1, 1).start()
    i_copy(0, 0).wait()
    g_copy(0).start()

    def pair_body(k2, _):
        for q in range(2):
            k = k2 * 2 + q
            g_copy(q).wait()

            @pl.when(k + 2 < nchunk)
            def _():
                i_copy(k + 2, q).start()

            @pl.when(k + 1 < nchunk)
            def _():
                i_copy(k + 1, 1 - q).wait()
                g_copy(1 - q).start()

            a_copy(k, q).wait()

            @pl.when(k >= 2)
            def _():
                s_copy(k - 2, q).wait()

            g = gat_v[q]
            av_ref = a_v[q]
            sv_ref = s_v[q]
            for a in range(CHUNK):
                for f in range(FEA // 16):
                    av = av_ref[a, pl.ds(f * 16, 16)]
                    acc = _round_bf16(jnp.maximum(
                        av + g[a * MAX_NBR, pl.ds(f * 16, 16)], 0.0))
                    for n in range(1, MAX_NBR):
                        acc = acc + _round_bf16(jnp.maximum(
                            av + g[a * MAX_NBR + n, pl.ds(f * 16, 16)], 0.0))
                    sv_ref[a, pl.ds(f * 16, 16)] = acc
            s_copy(k, q).start()

            @pl.when(k + 2 < nchunk)
            def _():
                a_copy(k + 2, q).start()
        return ()

    lax.fori_loop(0, nchunk // 2, pair_body, ())
    s_copy(nchunk - 2, 0).wait()
    s_copy(nchunk - 1, 1).wait()


@jax.jit
def _sc_segment_relu_sum(a_pad, b_pad, idx2):
    run = pl.kernel(
        _sc_body,
        out_type=jax.ShapeDtypeStruct((N_PAD, FEA), F32),
        mesh=plsc.VectorSubcoreMesh(core_axis_name="c", subcore_axis_name="s"),
        scratch_types=[
            pltpu.VMEM((CHUNK * MAX_NBR,), jnp.int32),
            pltpu.VMEM((CHUNK * MAX_NBR,), jnp.int32),
            pltpu.VMEM((CHUNK * MAX_NBR, FEA), F32),
            pltpu.VMEM((CHUNK * MAX_NBR, FEA), F32),
            pltpu.VMEM((CHUNK, FEA), F32),
            pltpu.VMEM((CHUNK, FEA), F32),
            pltpu.VMEM((CHUNK, FEA), F32),
            pltpu.VMEM((CHUNK, FEA), F32),
        ] + [pltpu.SemaphoreType.DMA] * 8,
        compiler_params=pltpu.CompilerParams(use_tc_tiling_on_sc=False,
                                             needs_layout_passes=False),
    )
    return run(a_pad, b_pad, idx2)


def _full(shape):
    return pl.BlockSpec(shape, lambda i: (0, 0))


def _rows(cols):
    return pl.BlockSpec((ROW_BLK, cols), lambda i: (i, 0))


def _dot(x, w):
    return jnp.dot(x.astype(jnp.bfloat16), w.astype(jnp.bfloat16),
                   preferred_element_type=F32)


def _dot_f32(x, w):
    return jnp.dot(x, w, preferred_element_type=F32,
                   precision=lax.Precision.HIGHEST)


def _stage_embed(af_pad, we, be, w1c, b1c):
    def body(af_ref, we_ref, be_ref, w1_ref, b1_ref, x_ref, a_ref, b_ref):
        x = _dot(af_ref[...], we_ref[...]) + be_ref[...]
        x_ref[...] = x
        ab = _dot(x, w1_ref[...]) + b1_ref[...]
        a_ref[...] = ab[:, :FEA]
        b_ref[...] = ab[:, FEA:]

    return pl.pallas_call(
        body,
        grid=(N_TC_BLOCKS,),
        in_specs=[_rows(128), _full((128, FEA)), _full((1, FEA)),
                  _full((FEA, 2 * FEA)), _full((1, 2 * FEA))],
        out_specs=[_rows(FEA), _rows(FEA), _rows(FEA)],
        out_shape=[jax.ShapeDtypeStruct((N_PAD, FEA), F32),
                   jax.ShapeDtypeStruct((N_PAD, FEA), F32),
                   jax.ShapeDtypeStruct((N_PAD, FEA), F32)],
    )(af_pad, we, be, w1c, b1c)


def _update_block(x, s, w2_ref, b2s_ref, u1a_ref, u1b_ref, bu1_ref,
                  u2_ref, bu2_ref):
    agg = _dot_f32(s, w2_ref[...]) + b2s_ref[...]
    t = jax.nn.relu(_dot(x, u1a_ref[...]) + _dot(agg, u1b_ref[...])
                    + bu1_ref[...])
    upd = _dot(t, u2_ref[...]) + bu2_ref[...]
    return jax.nn.relu(x + upd)


def _stage_update(x, s, w2, b2s, u1a, u1b, bu1, u2, bu2, w1c, b1c):
    def body(x_ref, s_ref, w2_ref, b2s_ref, u1a_ref, u1b_ref, bu1_ref,
             u2_ref, bu2_ref, w1_ref, b1_ref, xn_ref, a_ref, b_ref):
        xn = _update_block(x_ref[...], s_ref[...], w2_ref, b2s_ref,
                           u1a_ref, u1b_ref, bu1_ref, u2_ref, bu2_ref)
        xn_ref[...] = xn
        ab = _dot(xn, w1_ref[...]) + b1_ref[...]
        a_ref[...] = ab[:, :FEA]
        b_ref[...] = ab[:, FEA:]

    wspec = _full((FEA, FEA))
    bspec = _full((1, FEA))
    return pl.pallas_call(
        body,
        grid=(N_TC_BLOCKS,),
        in_specs=[_rows(FEA), _rows(FEA), wspec, bspec, wspec, wspec, bspec,
                  wspec, bspec, _full((FEA, 2 * FEA)), _full((1, 2 * FEA))],
        out_specs=[_rows(FEA), _rows(FEA), _rows(FEA)],
        out_shape=[jax.ShapeDtypeStruct((N_PAD, FEA), F32),
                   jax.ShapeDtypeStruct((N_PAD, FEA), F32),
                   jax.ShapeDtypeStruct((N_PAD, FEA), F32)],
    )(x, s, w2, b2s, u1a, u1b, bu1, u2, bu2, w1c, b1c)


def _stage_final(x, s, w2, b2s, u1a, u1b, bu1, u2, bu2,
                 f1, bf1, f2, bf2, hw, bh):
    def body(x_ref, s_ref, w2_ref, b2s_ref, u1a_ref, u1b_ref, bu1_ref,
             u2_ref, bu2_ref, f1_ref, bf1_ref, f2_ref, bf2_ref,
             hw_ref, bh_ref, xn_ref, feat_ref, prop_ref):
        xn = _update_block(x_ref[...], s_ref[...], w2_ref, b2s_ref,
                           u1a_ref, u1b_ref, bu1_ref, u2_ref, bu2_ref)
        xn_ref[...] = xn
        h = jax.nn.relu(_dot(xn, f1_ref[...]) + bf1_ref[...])
        feat = _dot(h, f2_ref[...]) + bf2_ref[...]
        feat_ref[...] = feat
        prop_ref[...] = _dot(feat, hw_ref[...]) + bh_ref[...]

    wspec = _full((FEA, FEA))
    bspec = _full((1, FEA))
    return pl.pallas_call(
        body,
        grid=(N_TC_BLOCKS,),
        in_specs=[_rows(FEA), _rows(FEA), wspec, bspec, wspec, wspec, bspec,
                  wspec, bspec, _full((FEA, 2 * FEA)), _full((1, 2 * FEA)),
                  _full((2 * FEA, FEA)), _full((1, FEA)),
                  _full((FEA, 8)), _full((1, 8))],
        out_specs=[_rows(FEA), _rows(FEA), _rows(8)],
        out_shape=[jax.ShapeDtypeStruct((N_PAD, FEA), F32),
                   jax.ShapeDtypeStruct((N_PAD, FEA), F32),
                   jax.ShapeDtypeStruct((N_PAD, 8), F32)],
    )(x, s, w2, b2s, u1a, u1b, bu1, u2, bu2, f1, bf1, f2, bf2, hw, bh)


def _conv_weights(cp):
    w1, b1 = cp["msg1"]
    w2, b2 = cp["msg2"]
    u1, bu1 = cp["upd1"]
    u2, bu2 = cp["upd2"]
    w1c = jnp.concatenate([w1[:FEA], w1[FEA:]], axis=1)
    b1c = jnp.concatenate([b1, jnp.zeros((FEA,), F32)])[None, :]
    b2s = (MAX_NBR * b2)[None, :]
    w2r = w2.astype(jnp.bfloat16).astype(F32)
    return (w1c, b1c, w2r, b2s, u1[:FEA], u1[FEA:], bu1[None, :],
            u2, bu2[None, :])


def kernel(atom_fea, nbr_fea, nbr_fea_idx, params):
    del nbr_fea

    n = atom_fea.shape[0]
    pad = N_PAD - n

    af = jnp.pad(atom_fea, ((0, pad), (0, 128 - atom_fea.shape[1])))
    idx2 = jnp.pad(nbr_fea_idx, ((0, pad), (0, 0))).reshape(
        N_PAD * MAX_NBR // (CHUNK * MAX_NBR), CHUNK * MAX_NBR)

    we, be = params["atom_emb"]
    we = jnp.pad(we, ((0, 128 - we.shape[0]), (0, 0)))
    conv_ws = [_conv_weights(cp) for cp in params["convs"]]

    f1, bf1 = params["feat1"]
    f2, bf2 = params["feat2"]
    head_w = jnp.concatenate(
        [params[nm][0] for nm in
         ("bond_strength", "local_charge", "coordination_energy",
          "migration_barrier")], axis=1)
    head_b = jnp.stack(
        [params[nm][1][0] for nm in
         ("bond_strength", "local_charge", "coordination_energy",
          "migration_barrier")])
    head_w = jnp.pad(head_w, ((0, 0), (0, 4)))
    head_b = jnp.pad(head_b, (0, 4))[None, :]

    w1c0, b1c0 = conv_ws[0][0], conv_ws[0][1]
    x, a_arr, b_arr = _stage_embed(af, we, be[None, :], w1c0, b1c0)

    for c in range(3):
        (w1c, b1c, w2, b2s, u1a, u1b, bu1, u2, bu2) = conv_ws[c]
        s = _sc_segment_relu_sum(a_arr, b_arr, idx2)
        if c < 2:
            nw1c, nb1c = conv_ws[c + 1][0], conv_ws[c + 1][1]
            x, a_arr, b_arr = _stage_update(x, s, w2, b2s, u1a, u1b, bu1,
                                            u2, bu2, nw1c, nb1c)
        else:
            x, feat, props = _stage_final(x, s, w2, b2s, u1a, u1b, bu1,
                                          u2, bu2, f1, bf1[None, :],
                                          f2, bf2[None, :], head_w, head_b)

    props = props[:n]
    return (props[:, 0:1], props[:, 1:2], props[:, 2:3], props[:, 3:4],
            feat[:n], x[:n])


# speedup vs baseline: 2.1544x; 1.0074x over previous
import functools

import jax
import jax.numpy as jnp
from jax import lax
from jax.experimental import pallas as pl
from jax.experimental.pallas import tpu as pltpu
from jax.experimental.pallas import tpu_sc as plsc

F32 = jnp.float32

N_ATOMS = 50000
MAX_NBR = 16
FEA = 64

NW = 32
CHUNK = 8
ROWS_C0 = 2176
ROWS_C1 = 1024
N_PAD = 16 * (ROWS_C0 + ROWS_C1)

ROW_BLK = 1024
N_TC_BLOCKS = N_PAD // ROW_BLK


def _round_bf16(v):
    u = plsc.bitcast(v, jnp.int32)
    r = (u + 0x7FFF + ((u >> 16) & 1)) & jnp.int32(-65536)
    return plsc.bitcast(r, F32)


def _sc_body(a_hbm, b_hbm, idx2_hbm, s_hbm,
             idx_v0, idx_v1, gat_v0, gat_v1, a_v0, a_v1, s_v0, s_v1,
             isem0, isem1, gsem0, gsem1, g2sem0, g2sem1,
             asem0, asem1, ssem0, ssem1):
    cid = lax.axis_index("c")
    sid = lax.axis_index("s")
    base = lax.select(cid == 0, sid * ROWS_C0,
                      16 * ROWS_C0 + sid * ROWS_C1)
    nchunk = lax.select(cid == 0, ROWS_C0 // CHUNK, ROWS_C1 // CHUNK)
    irow0 = base // CHUNK

    idx_v = (idx_v0, idx_v1)
    gat_v = (gat_v0, gat_v1)
    a_v = (a_v0, a_v1)
    s_v = (s_v0, s_v1)
    isem = (isem0, isem1)
    gsem = (gsem0, gsem1)
    gsem2 = (g2sem0, g2sem1)
    asem = (asem0, asem1)
    ssem = (ssem0, ssem1)

    def i_copy(k, p):
        return pltpu.make_async_copy(idx2_hbm.at[irow0 + k], idx_v[p], isem[p])

    half = CHUNK * MAX_NBR // 2

    class _GPair:
        def __init__(self, p):
            self.c1 = pltpu.make_async_copy(
                b_hbm.at[idx_v[p].at[pl.ds(0, half)]],
                gat_v[p].at[pl.ds(0, half)], gsem[p])
            self.c2 = pltpu.make_async_copy(
                b_hbm.at[idx_v[p].at[pl.ds(half, half)]],
                gat_v[p].at[pl.ds(half, half)], gsem2[p])

        def start(self):
            self.c1.start()
            self.c2.start()

        def wait(self):
            self.c1.wait()
            self.c2.wait()

    def g_copy(p):
        return _GPair(p)

    def a_copy(k, p):
        return pltpu.make_async_copy(a_hbm.at[pl.ds(base + k * CHUNK, CHUNK)],
                                     a_v[p], asem[p])

    def s_copy(k, p):
        return pltpu.make_async_copy(s_v[p],
                                     s_hbm.at[pl.ds(base + k * CHUNK, CHUNK)],
                                     ssem[p])

    i_copy(0, 0).start()
    i_copy(1, 1).start()
    a_copy(0, 0).start()
    a_copy(1, 1).start()
    i_copy(0, 0).wait()
    g_copy(0).start()

    def pair_body(k2, _):
        for q in range(2):
            k = k2 * 2 + q
            g_copy(q).wait()

            @pl.when(k + 2 < nchunk)
            def _():
                i_copy(k + 2, q).start()

            @pl.when(k + 1 < nchunk)
            def _():
                i_copy(k + 1, 1 - q).wait()
                g_copy(1 - q).start()

            a_copy(k, q).wait()

            @pl.when(k >= 2)
            def _():
                s_copy(k - 2, q).wait()

            g = gat_v[q]
            av_ref = a_v[q]
            sv_ref = s_v[q]
            for a in range(CHUNK):
                for f in range(FEA // 16):
                    av = av_ref[a, pl.ds(f * 16, 16)]
                    acc = _round_bf16(jnp.maximum(
                        av + g[a * MAX_NBR, pl.ds(f * 16, 16)], 0.0))
                    for n in range(1, MAX_NBR):
                        acc = acc + _round_bf16(jnp.maximum(
                            av + g[a * MAX_NBR + n, pl.ds(f * 16, 16)], 0.0))
                    sv_ref[a, pl.ds(f * 16, 16)] = acc
            s_copy(k, q).start()

            @pl.when(k + 2 < nchunk)
            def _():
                a_copy(k + 2, q).start()
        return ()

    lax.fori_loop(0, nchunk // 2, pair_body, ())
    s_copy(nchunk - 2, 0).wait()
    s_copy(nchunk - 1, 1).wait()


@jax.jit
def _sc_segment_relu_sum(a_pad, b_pad, idx2):
    run = pl.kernel(
        _sc_body,
        out_type=jax.ShapeDtypeStruct((N_PAD, FEA), F32),
        mesh=plsc.VectorSubcoreMesh(core_axis_name="c", subcore_axis_name="s"),
        scratch_types=[
            pltpu.VMEM((CHUNK * MAX_NBR,), jnp.int32),
            pltpu.VMEM((CHUNK * MAX_NBR,), jnp.int32),
            pltpu.VMEM((CHUNK * MAX_NBR, FEA), F32),
            pltpu.VMEM((CHUNK * MAX_NBR, FEA), F32),
            pltpu.VMEM((CHUNK, FEA), F32),
            pltpu.VMEM((CHUNK, FEA), F32),
            pltpu.VMEM((CHUNK, FEA), F32),
            pltpu.VMEM((CHUNK, FEA), F32),
        ] + [pltpu.SemaphoreType.DMA] * 10,
        compiler_params=pltpu.CompilerParams(use_tc_tiling_on_sc=False,
                                             needs_layout_passes=False),
    )
    return run(a_pad, b_pad, idx2)


def _full(shape):
    return pl.BlockSpec(shape, lambda i: (0, 0))


def _rows(cols):
    return pl.BlockSpec((ROW_BLK, cols), lambda i: (i, 0))


def _dot(x, w):
    return jnp.dot(x.astype(jnp.bfloat16), w.astype(jnp.bfloat16),
                   preferred_element_type=F32)


def _dot_f32(x, w):
    return jnp.dot(x, w, preferred_element_type=F32,
                   precision=lax.Precision.HIGHEST)


def _stage_embed(af_pad, we, be, w1c, b1c):
    def body(af_ref, we_ref, be_ref, w1_ref, b1_ref, x_ref, a_ref, b_ref):
        x = _dot(af_ref[...], we_ref[...]) + be_ref[...]
        x_ref[...] = x
        ab = _dot(x, w1_ref[...]) + b1_ref[...]
        a_ref[...] = ab[:, :FEA]
        b_ref[...] = ab[:, FEA:]

    return pl.pallas_call(
        body,
        grid=(N_TC_BLOCKS,),
        in_specs=[_rows(128), _full((128, FEA)), _full((1, FEA)),
                  _full((FEA, 2 * FEA)), _full((1, 2 * FEA))],
        out_specs=[_rows(FEA), _rows(FEA), _rows(FEA)],
        out_shape=[jax.ShapeDtypeStruct((N_PAD, FEA), F32),
                   jax.ShapeDtypeStruct((N_PAD, FEA), F32),
                   jax.ShapeDtypeStruct((N_PAD, FEA), F32)],
    )(af_pad, we, be, w1c, b1c)


def _update_block(x, s, w2_ref, b2s_ref, u1a_ref, u1b_ref, bu1_ref,
                  u2_ref, bu2_ref):
    agg = _dot_f32(s, w2_ref[...]) + b2s_ref[...]
    t = jax.nn.relu(_dot(x, u1a_ref[...]) + _dot(agg, u1b_ref[...])
                    + bu1_ref[...])
    upd = _dot(t, u2_ref[...]) + bu2_ref[...]
    return jax.nn.relu(x + upd)


def _stage_update(x, s, w2, b2s, u1a, u1b, bu1, u2, bu2, w1c, b1c):
    def body(x_ref, s_ref, w2_ref, b2s_ref, u1a_ref, u1b_ref, bu1_ref,
             u2_ref, bu2_ref, w1_ref, b1_ref, xn_ref, a_ref, b_ref):
        xn = _update_block(x_ref[...], s_ref[...], w2_ref, b2s_ref,
                           u1a_ref, u1b_ref, bu1_ref, u2_ref, bu2_ref)
        xn_ref[...] = xn
        ab = _dot(xn, w1_ref[...]) + b1_ref[...]
        a_ref[...] = ab[:, :FEA]
        b_ref[...] = ab[:, FEA:]

    wspec = _full((FEA, FEA))
    bspec = _full((1, FEA))
    return pl.pallas_call(
        body,
        grid=(N_TC_BLOCKS,),
        in_specs=[_rows(FEA), _rows(FEA), wspec, bspec, wspec, wspec, bspec,
                  wspec, bspec, _full((FEA, 2 * FEA)), _full((1, 2 * FEA))],
        out_specs=[_rows(FEA), _rows(FEA), _rows(FEA)],
        out_shape=[jax.ShapeDtypeStruct((N_PAD, FEA), F32),
                   jax.ShapeDtypeStruct((N_PAD, FEA), F32),
                   jax.ShapeDtypeStruct((N_PAD, FEA), F32)],
    )(x, s, w2, b2s, u1a, u1b, bu1, u2, bu2, w1c, b1c)


def _stage_final(x, s, w2, b2s, u1a, u1b, bu1, u2, bu2,
                 f1, bf1, f2, bf2, hw, bh):
    def body(x_ref, s_ref, w2_ref, b2s_ref, u1a_ref, u1b_ref, bu1_ref,
             u2_ref, bu2_ref, f1_ref, bf1_ref, f2_ref, bf2_ref,
             hw_ref, bh_ref, xn_ref, feat_ref, prop_ref):
        xn = _update_block(x_ref[...], s_ref[...], w2_ref, b2s_ref,
                           u1a_ref, u1b_ref, bu1_ref, u2_ref, bu2_ref)
        xn_ref[...] = xn
        h = jax.nn.relu(_dot(xn, f1_ref[...]) + bf1_ref[...])
        feat = _dot(h, f2_ref[...]) + bf2_ref[...]
        feat_ref[...] = feat
        prop_ref[...] = _dot(feat, hw_ref[...]) + bh_ref[...]

    wspec = _full((FEA, FEA))
    bspec = _full((1, FEA))
    return pl.pallas_call(
        body,
        grid=(N_TC_BLOCKS,),
        in_specs=[_rows(FEA), _rows(FEA), wspec, bspec, wspec, wspec, bspec,
                  wspec, bspec, _full((FEA, 2 * FEA)), _full((1, 2 * FEA)),
                  _full((2 * FEA, FEA)), _full((1, FEA)),
                  _full((FEA, 8)), _full((1, 8))],
        out_specs=[_rows(FEA), _rows(FEA), _rows(8)],
        out_shape=[jax.ShapeDtypeStruct((N_PAD, FEA), F32),
                   jax.ShapeDtypeStruct((N_PAD, FEA), F32),
                   jax.ShapeDtypeStruct((N_PAD, 8), F32)],
    )(x, s, w2, b2s, u1a, u1b, bu1, u2, bu2, f1, bf1, f2, bf2, hw, bh)


def _conv_weights(cp):
    w1, b1 = cp["msg1"]
    w2, b2 = cp["msg2"]
    u1, bu1 = cp["upd1"]
    u2, bu2 = cp["upd2"]
    w1c = jnp.concatenate([w1[:FEA], w1[FEA:]], axis=1)
    b1c = jnp.concatenate([b1, jnp.zeros((FEA,), F32)])[None, :]
    b2s = (MAX_NBR * b2)[None, :]
    w2r = w2.astype(jnp.bfloat16).astype(F32)
    return (w1c, b1c, w2r, b2s, u1[:FEA], u1[FEA:], bu1[None, :],
            u2, bu2[None, :])


def kernel(atom_fea, nbr_fea, nbr_fea_idx, params):
    del nbr_fea

    n = atom_fea.shape[0]
    pad = N_PAD - n

    af = jnp.pad(atom_fea, ((0, pad), (0, 128 - atom_fea.shape[1])))
    idx2 = jnp.pad(nbr_fea_idx, ((0, pad), (0, 0))).reshape(
        N_PAD * MAX_NBR // (CHUNK * MAX_NBR), CHUNK * MAX_NBR)

    we, be = params["atom_emb"]
    we = jnp.pad(we, ((0, 128 - we.shape[0]), (0, 0)))
    conv_ws = [_conv_weights(cp) for cp in params["convs"]]

    f1, bf1 = params["feat1"]
    f2, bf2 = params["feat2"]
    head_w = jnp.concatenate(
        [params[nm][0] for nm in
         ("bond_strength", "local_charge", "coordination_energy",
          "migration_barrier")], axis=1)
    head_b = jnp.stack(
        [params[nm][1][0] for nm in
         ("bond_strength", "local_charge", "coordination_energy",
          "migration_barrier")])
    head_w = jnp.pad(head_w, ((0, 0), (0, 4)))
    head_b = jnp.pad(head_b, (0, 4))[None, :]

    w1c0, b1c0 = conv_ws[0][0], conv_ws[0][1]
    x, a_arr, b_arr = _stage_embed(af, we, be[None, :], w1c0, b1c0)

    for c in range(3):
        (w1c, b1c, w2, b2s, u1a, u1b, bu1, u2, bu2) = conv_ws[c]
        s = _sc_segment_relu_sum(a_arr, b_arr, idx2)
        if c < 2:
            nw1c, nb1c = conv_ws[c + 1][0], conv_ws[c + 1][1]
            x, a_arr, b_arr = _stage_update(x, s, w2, b2s, u1a, u1b, bu1,
                                            u2, bu2, nw1c, nb1c)
        else:
            x, feat, props = _stage_final(x, s, w2, b2s, u1a, u1b, bu1,
                                          u2, bu2, f1, bf1[None, :],
                                          f2, bf2[None, :], head_w, head_b)

    props = props[:n]
    return (props[:, 0:1], props[:, 1:2], props[:, 2:3], props[:, 3:4],
            feat[:n], x[:n])
